# baseline (speedup 1.0000x reference)
import functools
import jax
import jax.numpy as jnp
from jax import lax
from jax.experimental import pallas as pl
from jax.experimental.pallas import tpu as pltpu
from jax.experimental.pallas import tpu_sc as plsc

N = 10000
E = 320000
F_IN = 128
H = 64
C = 10
G = 64

NC = 2
NS = 16
NW = NC * NS

NA = 10016
TROWS = NA // NS
DEG_PAD = 10240

EP_DEG = E // NW
NGRP_DEG = EP_DEG // 16
DEG_ROWS = DEG_PAD // 16
DEG_TROWS = DEG_ROWS // NS
DEG_CH = 128
DEG_NCH = DEG_ROWS // DEG_CH

K = 128
TOTCH = E // K
NCHF = 79
SLOWCH = TOTCH - NS * NCHF
NCHS = SLOWCH // NS
NREM = SLOWCH - NS * NCHS
NCHS4 = NCHS + 1
FASTCH = NS * NCHF
NB = 7
DEPTH = 3
NGRP = (NCHF + 2 * NB - 1) // NB

BR = 2048
NBLK = (NA + BR - 1) // BR

_SC_PARAMS = pltpu.CompilerParams(
    needs_layout_passes=False, use_tc_tiling_on_sc=False)


def _deg_body(dst_hbm, out_hbm, dstv, priv1, priv2):
    c = lax.axis_index("c")
    sid = lax.axis_index("s")
    wid = sid * NC + c

    pltpu.sync_copy(dst_hbm.at[wid], dstv)

    def zero_row(i, _):
        priv1[pl.ds(i * 16, 16)] = jnp.zeros((16,), jnp.float32)
        return _
    lax.fori_loop(0, DEG_ROWS, zero_row, None)

    ones = jnp.ones((16,), jnp.float32)

    def acc(i, _):
        plsc.addupdate_scatter(priv1, [dstv[i]], ones)
        return _
    lax.fori_loop(0, NGRP_DEG, acc, None)

    def to_rows(i, _):
        priv2[i, :] = priv1[pl.ds(i * 16, 16)]
        return _
    lax.fori_loop(0, DEG_ROWS, to_rows, None)

    pltpu.sync_copy(priv2, out_hbm.at[wid])


def _sc_degree(dst_grp):
    mesh = plsc.VectorSubcoreMesh(core_axis_name="c", subcore_axis_name="s")
    f = pl.kernel(
        _deg_body,
        out_type=jax.ShapeDtypeStruct((NW, DEG_ROWS, 16), jnp.float32),
        mesh=mesh,
        scratch_types=[
            pltpu.VMEM((NGRP_DEG, 16), jnp.int32),
            pltpu.VMEM((DEG_PAD,), jnp.float32),
            pltpu.VMEM((DEG_ROWS, 16), jnp.float32),
        ],
        compiler_params=_SC_PARAMS,
    )
    return f(dst_grp)


def _agg_body(y_hbm, src_hbm, dst_hbm, out_hbm, srcv, dstv, rows,
              agg_sh, gsem, ssem, isem):
    c = lax.axis_index("c")
    sid = lax.axis_index("s")

    base = jnp.where(
        c == 0, sid * NCHF,
        FASTCH + NCHS * sid + jnp.minimum(sid, NREM))
    cnt = jnp.where(c == 0, NCHF,
                    jnp.where(sid < NREM, NCHS4, NCHS))

    @pl.when(c == 0)
    def _load_fast():
        pltpu.async_copy(src_hbm.at[pl.ds(base, NCHF)], srcv, isem.at[0])
        pltpu.async_copy(dst_hbm.at[pl.ds(base, NCHF)], dstv, isem.at[1])

    @pl.when(jnp.logical_and(c == 1, sid < NREM))
    def _load_slow4():
        pltpu.async_copy(src_hbm.at[pl.ds(base, NCHS4)],
                         srcv.at[pl.ds(0, NCHS4)], isem.at[0])
        pltpu.async_copy(dst_hbm.at[pl.ds(base, NCHS4)],
                         dstv.at[pl.ds(0, NCHS4)], isem.at[1])

    @pl.when(jnp.logical_and(c == 1, sid >= NREM))
    def _load_slow():
        pltpu.async_copy(src_hbm.at[pl.ds(base, NCHS)],
                         srcv.at[pl.ds(0, NCHS)], isem.at[0])
        pltpu.async_copy(dst_hbm.at[pl.ds(base, NCHS)],
                         dstv.at[pl.ds(0, NCHS)], isem.at[1])

    def zero_row(i, _):
        for k4 in range(H // 16):
            rows[0, i, pl.ds(k4 * 16, 16)] = jnp.zeros((16,), jnp.float32)
        return _
    lax.fori_loop(0, K, zero_row, None)
    for z in range(TROWS // K):
        pltpu.sync_copy(rows.at[0],
                        agg_sh.at[pl.ds(sid * TROWS + z * K, K)])
    zr = TROWS - (TROWS // K) * K
    if zr:
        pltpu.sync_copy(rows.at[0, pl.ds(0, zr)],
                        agg_sh.at[pl.ds(sid * TROWS + (TROWS // K) * K, zr)])

    @pl.when(c == 0)
    def _wait_fast():
        pltpu.make_async_copy(src_hbm.at[pl.ds(base, NCHF)], srcv,
                              isem.at[0]).wait()
        pltpu.make_async_copy(dst_hbm.at[pl.ds(base, NCHF)], dstv,
                              isem.at[1]).wait()

    @pl.when(jnp.logical_and(c == 1, sid < NREM))
    def _wait_slow4():
        pltpu.make_async_copy(src_hbm.at[pl.ds(base, NCHS4)],
                              srcv.at[pl.ds(0, NCHS4)], isem.at[0]).wait()
        pltpu.make_async_copy(dst_hbm.at[pl.ds(base, NCHS4)],
                              dstv.at[pl.ds(0, NCHS4)], isem.at[1]).wait()

    @pl.when(jnp.logical_and(c == 1, sid >= NREM))
    def _wait_slow():
        pltpu.make_async_copy(src_hbm.at[pl.ds(base, NCHS)],
                              srcv.at[pl.ds(0, NCHS)], isem.at[0]).wait()
        pltpu.make_async_copy(dst_hbm.at[pl.ds(base, NCHS)],
                              dstv.at[pl.ds(0, NCHS)], isem.at[1]).wait()

    plsc.subcore_barrier()

    def group(g, _):
        for b in range(NB):
            j = g * NB + b

            @pl.when(jnp.logical_and(j >= NB, j - NB < cnt))
            def _drain():
                pltpu.make_async_copy(
                    rows.at[b], agg_sh.at[dstv.at[j - NB]],
                    ssem.at[b]).wait()

            @pl.when(j < cnt)
            def _gather():
                pltpu.async_copy(y_hbm.at[srcv.at[j]], rows.at[b],
                                 gsem.at[b])

            bd = (b - DEPTH) % NB

            @pl.when(jnp.logical_and(j >= DEPTH, j - DEPTH < cnt))
            def _scatter():
                jd = j - DEPTH
                pltpu.make_async_copy(y_hbm.at[srcv.at[jd]], rows.at[bd],
                                      gsem.at[bd]).wait()
                pltpu.async_copy(rows.at[bd], agg_sh.at[dstv.at[jd]],
                                 ssem.at[bd], add=True)
        return _
    lax.fori_loop(0, NGRP, group, None)

    plsc.subcore_barrier()

    pltpu.sync_copy(agg_sh.at[pl.ds(sid * TROWS, TROWS)],
                    out_hbm.at[c, pl.ds(sid * TROWS, TROWS)])


def _sc_agg(y_pad, src_grp, dst_grp):
    mesh = plsc.VectorSubcoreMesh(core_axis_name="c", subcore_axis_name="s")
    f = pl.kernel(
        _agg_body,
        out_type=jax.ShapeDtypeStruct((NC, NA, H), jnp.float32),
        mesh=mesh,
        scratch_types=[
            pltpu.VMEM((NCHF, K), jnp.int32),
            pltpu.VMEM((NCHF, K), jnp.int32),
            pltpu.VMEM((NB, K, H), jnp.float32),
            pltpu.VMEM_SHARED((NA, H), jnp.float32),
            pltpu.SemaphoreType.DMA((NB,)),
            pltpu.SemaphoreType.DMA((NB,)),
            pltpu.SemaphoreType.DMA((2,)),
        ],
        compiler_params=_SC_PARAMS,
    )
    return f(y_pad, src_grp, dst_grp)


def _dinv_body(p_ref, o_ref):
    d = jnp.sum(p_ref[...], axis=0) + 1.0
    r = lax.broadcasted_iota(jnp.int32, (DEG_PAD // 128, 128), 0)
    col = lax.broadcasted_iota(jnp.int32, (DEG_PAD // 128, 128), 1)
    idx = r * 128 + col
    o_ref[...] = jnp.where(idx < N, lax.rsqrt(d), 0.0)


def _tc_dinv(parts2):
    return pl.pallas_call(
        _dinv_body,
        out_shape=jax.ShapeDtypeStruct((DEG_PAD // 128, 128), jnp.float32),
    )(parts2)


def _first_body(x_ref, dinv_ref, w_ref, o_ref):
    o_ref[...] = dinv_ref[...] * jnp.dot(
        x_ref[...], w_ref[...], preferred_element_type=jnp.float32)


def _tc_first(x_pad, dinv, W1):
    return pl.pallas_call(
        _first_body,
        grid=(NBLK,),
        in_specs=[
            pl.BlockSpec((BR, F_IN), lambda i: (i, 0)),
            pl.BlockSpec((BR, 1), lambda i: (i, 0)),
            pl.BlockSpec((F_IN, H), lambda i: (0, 0)),
        ],
        out_specs=pl.BlockSpec((BR, H), lambda i: (i, 0)),
        out_shape=jax.ShapeDtypeStruct((NA, H), jnp.float32),
    )(x_pad, dinv, W1)


def _mid_body(p_ref, y_ref, dinv_ref, b_ref, w_ref, o_ref):
    dinv = dinv_ref[...]
    h = jax.nn.relu(dinv * (p_ref[0] + p_ref[1] + y_ref[...]) + b_ref[...])
    o_ref[...] = dinv * jnp.dot(h, w_ref[...],
                                preferred_element_type=jnp.float32)


def _tc_mid(parts, y, dinv, b, W):
    return pl.pallas_call(
        _mid_body,
        grid=(NBLK,),
        in_specs=[
            pl.BlockSpec((NC, BR, H), lambda i: (0, i, 0)),
            pl.BlockSpec((BR, H), lambda i: (i, 0)),
            pl.BlockSpec((BR, 1), lambda i: (i, 0)),
            pl.BlockSpec((1, H), lambda i: (0, 0)),
            pl.BlockSpec((H, H), lambda i: (0, 0)),
        ],
        out_specs=pl.BlockSpec((BR, H), lambda i: (i, 0)),
        out_shape=jax.ShapeDtypeStruct((NA, H), jnp.float32),
    )(parts, y, dinv, b, W)


def _pool_body(p_ref, y_ref, dinv_ref, b_ref, batch_ref, fcw_ref, fcb_ref,
               o_ref, sums_ref, cnts_ref):
    i = pl.program_id(0)
    dinv = dinv_ref[...]
    h = jax.nn.relu(dinv * (p_ref[0] + p_ref[1] + y_ref[...]) + b_ref[...])
    rowh = i * BR + lax.broadcasted_iota(jnp.int32, (BR, 1), 0)
    h = jnp.where(rowh < N, h, 0.0)
    gidx = lax.broadcasted_iota(jnp.int32, (BR, G), 1)
    rowid = i * BR + lax.broadcasted_iota(jnp.int32, (BR, G), 0)
    onehot = jnp.where((batch_ref[...] == gidx) & (rowid < N), 1.0, 0.0)
    sums_t = lax.dot_general(h, onehot, (((0,), (0,)), ((), ())),
                             preferred_element_type=jnp.float32)
    cnts = jnp.sum(onehot, axis=0, keepdims=True)

    @pl.when(i == 0)
    def _():
        sums_ref[...] = sums_t
        cnts_ref[...] = cnts

    @pl.when(i > 0)
    def _():
        sums_ref[...] += sums_t
        cnts_ref[...] += cnts

    @pl.when(i == NBLK - 1)
    def _():
        pooled_t = sums_ref[...] / jnp.maximum(cnts_ref[...], 1.0)
        logits = lax.dot_general(pooled_t, fcw_ref[...],
                                 (((0,), (0,)), ((), ())),
                                 preferred_element_type=jnp.float32)
        logits = logits + fcb_ref[...]
        m = jnp.max(logits, axis=1, keepdims=True)
        lse = m + jnp.log(jnp.sum(jnp.exp(logits - m), axis=1,
                                  keepdims=True))
        o_ref[...] = logits - lse


def _tc_pool(parts, y, dinv, b, batch2, fcW, fcb2):
    return pl.pallas_call(
        _pool_body,
        grid=(NBLK,),
        in_specs=[
            pl.BlockSpec((NC, BR, H), lambda i: (0, i, 0)),
            pl.BlockSpec((BR, H), lambda i: (i, 0)),
            pl.BlockSpec((BR, 1), lambda i: (i, 0)),
            pl.BlockSpec((1, H), lambda i: (0, 0)),
            pl.BlockSpec((BR, 1), lambda i: (i, 0)),
            pl.BlockSpec((H, C), lambda i: (0, 0)),
            pl.BlockSpec((1, C), lambda i: (0, 0)),
        ],
        out_specs=pl.BlockSpec((G, C), lambda i: (0, 0)),
        out_shape=jax.ShapeDtypeStruct((G, C), jnp.float32),
        scratch_shapes=[
            pltpu.VMEM((H, G), jnp.float32),
            pltpu.VMEM((1, G), jnp.float32),
        ],
    )(parts, y, dinv, b, batch2, fcW, fcb2)


def kernel(x, edge_index, batch, W1, b1, W2, b2, W3, b3, fcW, fcb):
    src, dst = edge_index[0], edge_index[1]

    dst_grp_deg = dst.reshape(NW, NGRP_DEG, 16)
    deg_parts = _sc_degree(dst_grp_deg)
    dinv = _tc_dinv(
        deg_parts.reshape(NW, DEG_PAD // 128, 128)).reshape(DEG_PAD, 1)
    dinv = dinv[:NA]

    src_grp = src.reshape(TOTCH, K)
    dst_grp = dst.reshape(TOTCH, K)
    batch2 = batch.reshape(N, 1)

    y = _tc_first(x, dinv, W1)
    parts = _sc_agg(y, src_grp, dst_grp)
    y = _tc_mid(parts, y, dinv, b1.reshape(1, H), W2)
    parts = _sc_agg(y, src_grp, dst_grp)
    y = _tc_mid(parts, y, dinv, b2.reshape(1, H), W3)
    parts = _sc_agg(y, src_grp, dst_grp)
    return _tc_pool(parts, y, dinv, b3.reshape(1, H), batch2, fcW,
                    fcb.reshape(1, C))


# speedup vs baseline: 1.0372x; 1.0372x over previous
import functools
import jax
import jax.numpy as jnp
from jax import lax
from jax.experimental import pallas as pl
from jax.experimental.pallas import tpu as pltpu
from jax.experimental.pallas import tpu_sc as plsc

N = 10000
E = 320000
F_IN = 128
H = 64
C = 10
G = 64

NC = 2
NS = 16
NW = NC * NS

NA = 10016
TROWS = NA // NS
DEG_PAD = 10240

EP_DEG = E // NW
NGRP_DEG = EP_DEG // 16
DEG_ROWS = DEG_PAD // 16
DEG_TROWS = DEG_ROWS // NS
DEG_CH = 128
DEG_NCH = DEG_ROWS // DEG_CH

K = 128
TOTCH = E // K
NCHF = 80
SLOWCH = TOTCH - NS * NCHF
NCHS = SLOWCH // NS
NREM = SLOWCH - NS * NCHS
NCHS4 = NCHS + 1
FASTCH = NS * NCHF
NB = 6
DEPTH = 3
NGRP = (NCHF + 2 * NB - 1) // NB

BR = 2048
NBLK = (NA + BR - 1) // BR

_SC_PARAMS = pltpu.CompilerParams(
    needs_layout_passes=False, use_tc_tiling_on_sc=False)


def _deg_body(dst_hbm, out_hbm, dstv, priv1, priv2):
    c = lax.axis_index("c")
    sid = lax.axis_index("s")
    wid = sid * NC + c

    pltpu.sync_copy(dst_hbm.at[wid], dstv)

    def zero_row(i, _):
        priv1[pl.ds(i * 16, 16)] = jnp.zeros((16,), jnp.float32)
        return _
    lax.fori_loop(0, DEG_ROWS, zero_row, None)

    ones = jnp.ones((16,), jnp.float32)

    def acc(i, _):
        plsc.addupdate_scatter(priv1, [dstv[i]], ones)
        return _
    lax.fori_loop(0, NGRP_DEG, acc, None)

    def to_rows(i, _):
        priv2[i, :] = priv1[pl.ds(i * 16, 16)]
        return _
    lax.fori_loop(0, DEG_ROWS, to_rows, None)

    pltpu.sync_copy(priv2, out_hbm.at[wid])


def _sc_degree(dst_grp):
    mesh = plsc.VectorSubcoreMesh(core_axis_name="c", subcore_axis_name="s")
    f = pl.kernel(
        _deg_body,
        out_type=jax.ShapeDtypeStruct((NW, DEG_ROWS, 16), jnp.float32),
        mesh=mesh,
        scratch_types=[
            pltpu.VMEM((NGRP_DEG, 16), jnp.int32),
            pltpu.VMEM((DEG_PAD,), jnp.float32),
            pltpu.VMEM((DEG_ROWS, 16), jnp.float32),
        ],
        compiler_params=_SC_PARAMS,
    )
    return f(dst_grp)


def _agg_body(y_hbm, src_hbm, dst_hbm, out_hbm, srcv, dstv, rows,
              agg_sh, gsem, ssem, isem):
    c = lax.axis_index("c")
    sid = lax.axis_index("s")

    base = jnp.where(
        c == 0, sid * NCHF,
        FASTCH + NCHS * sid + jnp.minimum(sid, NREM))
    cnt = jnp.where(c == 0, NCHF,
                    jnp.where(sid < NREM, NCHS4, NCHS))

    @pl.when(c == 0)
    def _load_fast():
        pltpu.async_copy(src_hbm.at[pl.ds(base, NCHF)], srcv, isem.at[0])
        pltpu.async_copy(dst_hbm.at[pl.ds(base, NCHF)], dstv, isem.at[1])

    @pl.when(jnp.logical_and(c == 1, sid < NREM))
    def _load_slow4():
        pltpu.async_copy(src_hbm.at[pl.ds(base, NCHS4)],
                         srcv.at[pl.ds(0, NCHS4)], isem.at[0])
        pltpu.async_copy(dst_hbm.at[pl.ds(base, NCHS4)],
                         dstv.at[pl.ds(0, NCHS4)], isem.at[1])

    @pl.when(jnp.logical_and(c == 1, sid >= NREM))
    def _load_slow():
        pltpu.async_copy(src_hbm.at[pl.ds(base, NCHS)],
                         srcv.at[pl.ds(0, NCHS)], isem.at[0])
        pltpu.async_copy(dst_hbm.at[pl.ds(base, NCHS)],
                         dstv.at[pl.ds(0, NCHS)], isem.at[1])

    def zero_row(i, _):
        for k4 in range(H // 16):
            rows[0, i, pl.ds(k4 * 16, 16)] = jnp.zeros((16,), jnp.float32)
        return _
    lax.fori_loop(0, K, zero_row, None)
    for z in range(TROWS // K):
        pltpu.sync_copy(rows.at[0],
                        agg_sh.at[pl.ds(sid * TROWS + z * K, K)])
    zr = TROWS - (TROWS // K) * K
    if zr:
        pltpu.sync_copy(rows.at[0, pl.ds(0, zr)],
                        agg_sh.at[pl.ds(sid * TROWS + (TROWS // K) * K, zr)])

    @pl.when(c == 0)
    def _wait_fast():
        pltpu.make_async_copy(src_hbm.at[pl.ds(base, NCHF)], srcv,
                              isem.at[0]).wait()
        pltpu.make_async_copy(dst_hbm.at[pl.ds(base, NCHF)], dstv,
                              isem.at[1]).wait()

    @pl.when(jnp.logical_and(c == 1, sid < NREM))
    def _wait_slow4():
        pltpu.make_async_copy(src_hbm.at[pl.ds(base, NCHS4)],
                              srcv.at[pl.ds(0, NCHS4)], isem.at[0]).wait()
        pltpu.make_async_copy(dst_hbm.at[pl.ds(base, NCHS4)],
                              dstv.at[pl.ds(0, NCHS4)], isem.at[1]).wait()

    @pl.when(jnp.logical_and(c == 1, sid >= NREM))
    def _wait_slow():
        pltpu.make_async_copy(src_hbm.at[pl.ds(base, NCHS)],
                              srcv.at[pl.ds(0, NCHS)], isem.at[0]).wait()
        pltpu.make_async_copy(dst_hbm.at[pl.ds(base, NCHS)],
                              dstv.at[pl.ds(0, NCHS)], isem.at[1]).wait()

    plsc.subcore_barrier()

    def group(g, _):
        for b in range(NB):
            j = g * NB + b

            @pl.when(jnp.logical_and(j >= NB, j - NB < cnt))
            def _drain():
                pltpu.make_async_copy(
                    rows.at[b], agg_sh.at[dstv.at[j - NB]],
                    ssem.at[b]).wait()

            @pl.when(j < cnt)
            def _gather():
                pltpu.async_copy(y_hbm.at[srcv.at[j]], rows.at[b],
                                 gsem.at[b])

            bd = (b - DEPTH) % NB

            @pl.when(jnp.logical_and(j >= DEPTH, j - DEPTH < cnt))
            def _scatter():
                jd = j - DEPTH
                pltpu.make_async_copy(y_hbm.at[srcv.at[jd]], rows.at[bd],
                                      gsem.at[bd]).wait()
                pltpu.async_copy(rows.at[bd], agg_sh.at[dstv.at[jd]],
                                 ssem.at[bd], add=True)
        return _
    lax.fori_loop(0, NGRP, group, None)

    plsc.subcore_barrier()

    pltpu.sync_copy(agg_sh.at[pl.ds(sid * TROWS, TROWS)],
                    out_hbm.at[c, pl.ds(sid * TROWS, TROWS)])


def _sc_agg(y_pad, src_grp, dst_grp):
    mesh = plsc.VectorSubcoreMesh(core_axis_name="c", subcore_axis_name="s")
    f = pl.kernel(
        _agg_body,
        out_type=jax.ShapeDtypeStruct((NC, NA, H), jnp.float32),
        mesh=mesh,
        scratch_types=[
            pltpu.VMEM((NCHF, K), jnp.int32),
            pltpu.VMEM((NCHF, K), jnp.int32),
            pltpu.VMEM((NB, K, H), jnp.float32),
            pltpu.VMEM_SHARED((NA, H), jnp.float32),
            pltpu.SemaphoreType.DMA((NB,)),
            pltpu.SemaphoreType.DMA((NB,)),
            pltpu.SemaphoreType.DMA((2,)),
        ],
        compiler_params=_SC_PARAMS,
    )
    return f(y_pad, src_grp, dst_grp)


def _dinv_body(p_ref, o_ref):
    d = jnp.sum(p_ref[...], axis=0) + 1.0
    r = lax.broadcasted_iota(jnp.int32, (DEG_PAD // 128, 128), 0)
    col = lax.broadcasted_iota(jnp.int32, (DEG_PAD // 128, 128), 1)
    idx = r * 128 + col
    o_ref[...] = jnp.where(idx < N, lax.rsqrt(d), 0.0)


def _tc_dinv(parts2):
    return pl.pallas_call(
        _dinv_body,
        out_shape=jax.ShapeDtypeStruct((DEG_PAD // 128, 128), jnp.float32),
    )(parts2)


def _first_body(x_ref, dinv_ref, w_ref, o_ref):
    o_ref[...] = dinv_ref[...] * jnp.dot(
        x_ref[...], w_ref[...], preferred_element_type=jnp.float32)


def _tc_first(x_pad, dinv, W1):
    return pl.pallas_call(
        _first_body,
        grid=(NBLK,),
        in_specs=[
            pl.BlockSpec((BR, F_IN), lambda i: (i, 0)),
            pl.BlockSpec((BR, 1), lambda i: (i, 0)),
            pl.BlockSpec((F_IN, H), lambda i: (0, 0)),
        ],
        out_specs=pl.BlockSpec((BR, H), lambda i: (i, 0)),
        out_shape=jax.ShapeDtypeStruct((NA, H), jnp.float32),
    )(x_pad, dinv, W1)


def _mid_body(p_ref, y_ref, dinv_ref, b_ref, w_ref, o_ref):
    dinv = dinv_ref[...]
    h = jax.nn.relu(dinv * (p_ref[0] + p_ref[1] + y_ref[...]) + b_ref[...])
    o_ref[...] = dinv * jnp.dot(h, w_ref[...],
                                preferred_element_type=jnp.float32)


def _tc_mid(parts, y, dinv, b, W):
    return pl.pallas_call(
        _mid_body,
        grid=(NBLK,),
        in_specs=[
            pl.BlockSpec((NC, BR, H), lambda i: (0, i, 0)),
            pl.BlockSpec((BR, H), lambda i: (i, 0)),
            pl.BlockSpec((BR, 1), lambda i: (i, 0)),
            pl.BlockSpec((1, H), lambda i: (0, 0)),
            pl.BlockSpec((H, H), lambda i: (0, 0)),
        ],
        out_specs=pl.BlockSpec((BR, H), lambda i: (i, 0)),
        out_shape=jax.ShapeDtypeStruct((NA, H), jnp.float32),
    )(parts, y, dinv, b, W)


def _pool_body(p_ref, y_ref, dinv_ref, b_ref, batch_ref, fcw_ref, fcb_ref,
               o_ref, sums_ref, cnts_ref):
    i = pl.program_id(0)
    dinv = dinv_ref[...]
    h = jax.nn.relu(dinv * (p_ref[0] + p_ref[1] + y_ref[...]) + b_ref[...])
    rowh = i * BR + lax.broadcasted_iota(jnp.int32, (BR, 1), 0)
    h = jnp.where(rowh < N, h, 0.0)
    gidx = lax.broadcasted_iota(jnp.int32, (BR, G), 1)
    rowid = i * BR + lax.broadcasted_iota(jnp.int32, (BR, G), 0)
    onehot = jnp.where((batch_ref[...] == gidx) & (rowid < N), 1.0, 0.0)
    sums_t = lax.dot_general(h, onehot, (((0,), (0,)), ((), ())),
                             preferred_element_type=jnp.float32)
    cnts = jnp.sum(onehot, axis=0, keepdims=True)

    @pl.when(i == 0)
    def _():
        sums_ref[...] = sums_t
        cnts_ref[...] = cnts

    @pl.when(i > 0)
    def _():
        sums_ref[...] += sums_t
        cnts_ref[...] += cnts

    @pl.when(i == NBLK - 1)
    def _():
        pooled_t = sums_ref[...] / jnp.maximum(cnts_ref[...], 1.0)
        logits = lax.dot_general(pooled_t, fcw_ref[...],
                                 (((0,), (0,)), ((), ())),
                                 preferred_element_type=jnp.float32)
        logits = logits + fcb_ref[...]
        m = jnp.max(logits, axis=1, keepdims=True)
        lse = m + jnp.log(jnp.sum(jnp.exp(logits - m), axis=1,
                                  keepdims=True))
        o_ref[...] = logits - lse


def _tc_pool(parts, y, dinv, b, batch2, fcW, fcb2):
    return pl.pallas_call(
        _pool_body,
        grid=(NBLK,),
        in_specs=[
            pl.BlockSpec((NC, BR, H), lambda i: (0, i, 0)),
            pl.BlockSpec((BR, H), lambda i: (i, 0)),
            pl.BlockSpec((BR, 1), lambda i: (i, 0)),
            pl.BlockSpec((1, H), lambda i: (0, 0)),
            pl.BlockSpec((BR, 1), lambda i: (i, 0)),
            pl.BlockSpec((H, C), lambda i: (0, 0)),
            pl.BlockSpec((1, C), lambda i: (0, 0)),
        ],
        out_specs=pl.BlockSpec((G, C), lambda i: (0, 0)),
        out_shape=jax.ShapeDtypeStruct((G, C), jnp.float32),
        scratch_shapes=[
            pltpu.VMEM((H, G), jnp.float32),
            pltpu.VMEM((1, G), jnp.float32),
        ],
    )(parts, y, dinv, b, batch2, fcW, fcb2)


def kernel(x, edge_index, batch, W1, b1, W2, b2, W3, b3, fcW, fcb):
    src, dst = edge_index[0], edge_index[1]

    dst_grp_deg = dst.reshape(NW, NGRP_DEG, 16)
    deg_parts = _sc_degree(dst_grp_deg)
    dinv = _tc_dinv(
        deg_parts.reshape(NW, DEG_PAD // 128, 128)).reshape(DEG_PAD, 1)
    dinv = dinv[:NA]

    src_grp = src.reshape(TOTCH, K)
    dst_grp = dst.reshape(TOTCH, K)
    batch2 = batch.reshape(N, 1)

    y = _tc_first(x, dinv, W1)
    parts = _sc_agg(y, src_grp, dst_grp)
    y = _tc_mid(parts, y, dinv, b1.reshape(1, H), W2)
    parts = _sc_agg(y, src_grp, dst_grp)
    y = _tc_mid(parts, y, dinv, b2.reshape(1, H), W3)
    parts = _sc_agg(y, src_grp, dst_grp)
    return _tc_pool(parts, y, dinv, b3.reshape(1, H), batch2, fcW,
                    fcb.reshape(1, C))
